# baseline (speedup 1.0000x reference)
import jax
import jax.numpy as jnp
from jax.experimental import pallas as pl
from jax.experimental.pallas import tpu as pltpu

N_NODES = 10000
N_EDGES = 320000
D_FEAT = 128
D_EDGE = 16
D_GLOBAL = 128

NUM_BLOCKS = 25
E_ROWS = N_EDGES * D_EDGE // 128
BE = E_ROWS // NUM_BLOCKS
BN = N_NODES // NUM_BLOCKS


def _body(edge_ref, node_ref, global_ref, w1_ref, b1_ref, w2_ref, b2_ref,
          out_ref, acc_e_ref, acc_n_ref):
    i = pl.program_id(0)

    @pl.when(i == 0)
    def _init():
        acc_e_ref[...] = jnp.zeros_like(acc_e_ref)
        acc_n_ref[...] = jnp.zeros_like(acc_n_ref)

    acc_e_ref[...] += jnp.sum(edge_ref[...], axis=0, keepdims=True)
    acc_n_ref[...] += jnp.sum(node_ref[...], axis=0, keepdims=True)

    @pl.when(i == NUM_BLOCKS - 1)
    def _finish():
        fold_r = jax.lax.broadcasted_iota(jnp.int32, (128, D_EDGE), 0)
        fold_c = jax.lax.broadcasted_iota(jnp.int32, (128, D_EDGE), 1)
        fold = (fold_r % D_EDGE == fold_c).astype(jnp.float32)
        agg_e = jnp.dot(acc_e_ref[...], fold,
                        preferred_element_type=jnp.float32) * (1.0 / N_EDGES)
        agg_n = acc_n_ref[...] * (1.0 / N_NODES)
        g = global_ref[...]
        w1 = w1_ref[...]
        pre = (
            jnp.dot(g, w1[0:D_GLOBAL, :], preferred_element_type=jnp.float32)
            + jnp.dot(agg_e, w1[D_GLOBAL:D_GLOBAL + D_EDGE, :],
                      preferred_element_type=jnp.float32)
            + jnp.dot(agg_n, w1[D_GLOBAL + D_EDGE:, :],
                      preferred_element_type=jnp.float32)
            + b1_ref[...]
        )
        h = jnp.maximum(pre, 0.0)
        out_ref[...] = (
            jnp.dot(h, w2_ref[...], preferred_element_type=jnp.float32)
            + b2_ref[...]
        )


def kernel(node_attr, edge_index, edge_attr, global_attr, W1, b1, W2, b2):
    del edge_index
    b1_2d = b1.reshape(1, -1)
    b2_2d = b2.reshape(1, -1)
    edge_wide = edge_attr.reshape(E_ROWS, 128)
    return pl.pallas_call(
        _body,
        grid=(NUM_BLOCKS,),
        in_specs=[
            pl.BlockSpec((BE, 128), lambda i: (i, 0)),
            pl.BlockSpec((BN, D_FEAT), lambda i: (i, 0)),
            pl.BlockSpec((1, D_GLOBAL), lambda i: (0, 0)),
            pl.BlockSpec((D_GLOBAL + D_EDGE + D_FEAT, 32), lambda i: (0, 0)),
            pl.BlockSpec((1, 32), lambda i: (0, 0)),
            pl.BlockSpec((32, D_FEAT), lambda i: (0, 0)),
            pl.BlockSpec((1, D_FEAT), lambda i: (0, 0)),
        ],
        out_specs=pl.BlockSpec((1, D_FEAT), lambda i: (0, 0)),
        out_shape=jax.ShapeDtypeStruct((1, D_FEAT), jnp.float32),
        scratch_shapes=[
            pltpu.VMEM((1, 128), jnp.float32),
            pltpu.VMEM((1, D_FEAT), jnp.float32),
        ],
    )(edge_wide, node_attr, global_attr, W1, b1_2d, W2, b2_2d)


# speedup vs baseline: 7.5399x; 7.5399x over previous
import jax
import jax.numpy as jnp
from jax.experimental import pallas as pl
from jax.experimental.pallas import tpu as pltpu

N_NODES = 10000
N_EDGES = 320000
D_FEAT = 128
D_EDGE = 16
D_GLOBAL = 128

NUM_BLOCKS = 25
BE = N_EDGES // NUM_BLOCKS
BN = N_NODES // NUM_BLOCKS


def _body(edge_ref, node_ref, global_ref, w1_ref, b1_ref, w2_ref, b2_ref,
          out_ref, acc_e_ref, acc_n_ref):
    i = pl.program_id(0)

    @pl.when(i == 0)
    def _init():
        acc_e_ref[...] = jnp.zeros_like(acc_e_ref)
        acc_n_ref[...] = jnp.zeros_like(acc_n_ref)

    e = edge_ref[...]
    acc = acc_e_ref[...]
    for k in range(BE // 128):
        acc = acc + e[:, k * 128:(k + 1) * 128]
    acc_e_ref[...] = acc
    acc_n_ref[...] += jnp.sum(node_ref[...], axis=0, keepdims=True)

    @pl.when(i == NUM_BLOCKS - 1)
    def _finish():
        esum = jnp.sum(acc_e_ref[...], axis=1, keepdims=True)
        agg_n = acc_n_ref[...] * (1.0 / N_NODES)
        g = global_ref[...]
        w1 = w1_ref[...]
        h_e = jax.lax.dot_general(
            esum * (1.0 / N_EDGES), w1[D_GLOBAL:D_GLOBAL + D_EDGE, :],
            (((0,), (0,)), ((), ())),
            preferred_element_type=jnp.float32,
        )
        pre = (
            jnp.dot(g, w1[0:D_GLOBAL, :], preferred_element_type=jnp.float32)
            + h_e
            + jnp.dot(agg_n, w1[D_GLOBAL + D_EDGE:, :],
                      preferred_element_type=jnp.float32)
            + b1_ref[...]
        )
        h = jnp.maximum(pre, 0.0)
        out_ref[...] = (
            jnp.dot(h, w2_ref[...], preferred_element_type=jnp.float32)
            + b2_ref[...]
        )


def kernel(node_attr, edge_index, edge_attr, global_attr, W1, b1, W2, b2):
    del edge_index
    b1_2d = b1.reshape(1, -1)
    b2_2d = b2.reshape(1, -1)
    edge_t = edge_attr.T
    return pl.pallas_call(
        _body,
        grid=(NUM_BLOCKS,),
        in_specs=[
            pl.BlockSpec((D_EDGE, BE), lambda i: (0, i)),
            pl.BlockSpec((BN, D_FEAT), lambda i: (i, 0)),
            pl.BlockSpec((1, D_GLOBAL), lambda i: (0, 0)),
            pl.BlockSpec((D_GLOBAL + D_EDGE + D_FEAT, 32), lambda i: (0, 0)),
            pl.BlockSpec((1, 32), lambda i: (0, 0)),
            pl.BlockSpec((32, D_FEAT), lambda i: (0, 0)),
            pl.BlockSpec((1, D_FEAT), lambda i: (0, 0)),
        ],
        out_specs=pl.BlockSpec((1, D_FEAT), lambda i: (0, 0)),
        out_shape=jax.ShapeDtypeStruct((1, D_FEAT), jnp.float32),
        scratch_shapes=[
            pltpu.VMEM((D_EDGE, 128), jnp.float32),
            pltpu.VMEM((1, D_FEAT), jnp.float32),
        ],
    )(edge_t, node_attr, global_attr, W1, b1_2d, W2, b2_2d)
